# baseline (speedup 1.0000x reference)
import functools

import jax
import jax.numpy as jnp
from jax import lax
from jax.experimental import pallas as pl
from jax.experimental.pallas import tpu as pltpu
from jax.experimental.pallas import tpu_sc as plsc

_LAMBDA = 10.0
_TEMP = 5.0
_K = 100
_PK = 128
_ROWS = 16
_NEG = -1.0e30
_CAND_CAP = 32768


def _sortable_u32(x):
    b = plsc.bitcast(x, jnp.uint32)
    sign = jnp.uint32(0x80000000)
    return jnp.where(b >= sign, ~b, b | sign)


def _popcnt(m):
    return plsc.all_reduce_population_count(m)[0]


def _sc_topk(teacher2d, student_flat):
    n, v = teacher2d.shape
    nw = 32
    rows_per_w = n // nw
    nvreg = v // 16

    mesh = plsc.VectorSubcoreMesh(
        core_axis_name="c", subcore_axis_name="s", num_cores=2,
        num_subcores=16)

    @functools.partial(
        pl.kernel,
        out_type=[
            jax.ShapeDtypeStruct((n, _PK), jnp.float32),
            jax.ShapeDtypeStruct((n, _PK), jnp.float32),
        ],
        mesh=mesh,
        compiler_params=pltpu.CompilerParams(needs_layout_passes=False),
        scratch_types=[
            pltpu.VMEM((v,), jnp.float32),
            pltpu.VMEM((_CAND_CAP,), jnp.float32),
            pltpu.VMEM((_CAND_CAP,), jnp.int32),
            pltpu.VMEM((_PK,), jnp.float32),
            pltpu.VMEM((_PK,), jnp.int32),
            pltpu.VMEM((_PK,), jnp.float32),
            pltpu.SemaphoreType.DMA,
        ],
    )
    def topk_kernel(t_hbm, s_hbm, out_t_hbm, out_s_hbm,
                    trow, cand_v, cand_i, outt, outi, outs, sem):
        wid = lax.axis_index("s") * 2 + lax.axis_index("c")
        lane = lax.broadcasted_iota(jnp.int32, (16,), 0)

        def scan_row(probe):
            def body(i, offv):
                x = trow[pl.ds(i * 16, 16)]
                m = x >= probe
                pf = plsc.cumsum(jnp.where(m, 1, 0))
                idx = offv + pf - 1
                plsc.store_scatter(cand_v, [idx], x, mask=m)
                plsc.store_scatter(cand_i, [idx], lane + i * 16, mask=m)
                return offv + plsc.all_reduce_population_count(m)
            offv = lax.fori_loop(0, nvreg, body,
                                 jnp.zeros((16,), jnp.int32))
            return offv[0]

        def cnt_ge(mid, nv, c):
            def body(j, acc):
                x = cand_v[pl.ds(j * 16, 16)]
                valid = (lane + j * 16) < c
                u = jnp.where(valid, _sortable_u32(x), jnp.uint32(0))
                return acc + jnp.where(u >= mid, 1, 0)
            accv = lax.fori_loop(0, nv, body, jnp.zeros((16,), jnp.int32))
            return plsc.cumsum(accv)[15]

        def do_row(j, _):
            row = wid * rows_per_w + j
            pltpu.sync_copy(t_hbm.at[row], trow)

            c0 = scan_row(jnp.float32(2.45))

            def cond(st):
                return st[2] < _K

            def lower(st):
                probe, step, _ = st
                probe2 = probe - step
                return probe2, step * 4.0, scan_row(probe2)

            probe_f, _, c = lax.while_loop(
                cond, lower, (jnp.float32(2.45), jnp.float32(1.5), c0))

            nv = (c + 15) // 16

            def vmax_body(j, acc):
                x = cand_v[pl.ds(j * 16, 16)]
                valid = (lane + j * 16) < c
                return jnp.maximum(acc, jnp.where(valid, x, -3.4e38))

            mx_vec = lax.fori_loop(
                0, nv, vmax_body, jnp.full((16,), -3.4e38, jnp.float32))
            hi_init = _sortable_u32(plsc.cummax(mx_vec))[15] + jnp.uint32(1)
            lo_init = _sortable_u32(jnp.full((16,), probe_f, jnp.float32))[0]

            def bis_cond(st):
                lo, hi = st
                return hi - lo > jnp.uint32(1)

            def bis(st):
                lo, hi = st
                mid = lo + lax.shift_right_logical(hi - lo, jnp.uint32(1))
                ge = cnt_ge(mid, nv, c) >= _K
                return (jnp.where(ge, mid, lo), jnp.where(ge, hi, mid))

            tau, _ = lax.while_loop(bis_cond, bis, (lo_init, hi_init))
            n_gt = cnt_ge(tau + jnp.uint32(1), nv, c)
            r = _K - n_gt

            def emit(jj, st):
                ooffv, eqrunv = st
                x = cand_v[pl.ds(jj * 16, 16)]
                gi = cand_i[pl.ds(jj * 16, 16)]
                valid = (lane + jj * 16) < c
                u = jnp.where(valid, _sortable_u32(x), jnp.uint32(0))
                m_gt = u > tau
                m_eq = u == tau
                pe = plsc.cumsum(jnp.where(m_eq, 1, 0))
                sel = jnp.logical_or(
                    m_gt, jnp.logical_and(m_eq, pe + eqrunv <= r))
                pfs = plsc.cumsum(jnp.where(sel, 1, 0))
                idx = ooffv + pfs - 1
                plsc.store_scatter(outt, [idx], x, mask=sel)
                plsc.store_scatter(outi, [idx], gi + row * v, mask=sel)
                return (ooffv + plsc.all_reduce_population_count(sel),
                        eqrunv + plsc.all_reduce_population_count(m_eq))

            lax.fori_loop(0, nv, emit, (jnp.zeros((16,), jnp.int32),
                                        jnp.zeros((16,), jnp.int32)))

            p6 = lane + 96
            outt[pl.ds(96, 16)] = jnp.where(
                p6 < _K, outt[pl.ds(96, 16)], jnp.float32(_NEG))
            outi[pl.ds(96, 16)] = jnp.where(
                p6 < _K, outi[pl.ds(96, 16)], row * v)
            outt[pl.ds(112, 16)] = jnp.full((16,), _NEG, jnp.float32)
            outi[pl.ds(112, 16)] = jnp.full((16,), row * v, jnp.int32)

            pltpu.async_copy(s_hbm.at[outi], outs, sem).wait()
            outs[pl.ds(96, 16)] = jnp.where(
                p6 < _K, outs[pl.ds(96, 16)], jnp.float32(_NEG))
            outs[pl.ds(112, 16)] = jnp.full((16,), _NEG, jnp.float32)

            pltpu.sync_copy(outt, out_t_hbm.at[row])
            pltpu.sync_copy(outs, out_s_hbm.at[row])
            return 0

        lax.fori_loop(0, rows_per_w, do_row, 0)

    return topk_kernel(teacher2d, student_flat)


def _tc_ce_kernel(s_ref, lab_ref, out_ref):
    r, v = s_ref.shape
    s = s_ref[...]
    lab = lab_ref[0, 0, :]

    ms = jnp.max(s, axis=1, keepdims=True)
    sum_es = jnp.sum(jnp.exp(s - ms), axis=1, keepdims=True)
    lse = jnp.log(sum_es) + ms
    col = jax.lax.broadcasted_iota(jnp.int32, (r, v), 1)
    lab_logit = jnp.sum(
        jnp.where(col == lab[:, None], s, 0.0), axis=1, keepdims=True)
    nll_block = jnp.sum(lse - lab_logit)

    @pl.when(pl.program_id(0) == 0)
    def _():
        out_ref[...] = jnp.zeros_like(out_ref)

    lane = jax.lax.broadcasted_iota(jnp.int32, (1, 128), 1)
    out_ref[...] += jnp.where(lane == 0, nll_block, 0.0)


def _tc_kl_kernel(tkt_ref, tks_ref, out_ref):
    inv_t = 1.0 / _TEMP
    tk_t = tkt_ref[...]
    tk_s = tks_ref[...]
    m_t = jnp.max(tk_t, axis=1, keepdims=True)
    e = jnp.exp((tk_t - m_t) * inv_t)
    a = jnp.sum(e, axis=1, keepdims=True)
    m_s = jnp.max(tk_s, axis=1, keepdims=True)
    a_s = jnp.sum(jnp.exp((tk_s - m_s) * inv_t), axis=1, keepdims=True)
    u_sum = jnp.sum(e * (tk_t - tk_s), axis=1, keepdims=True) * inv_t
    kl_row = u_sum / a + (m_s - m_t) * inv_t + jnp.log(a_s) - jnp.log(a)
    kl_block = jnp.sum(kl_row)

    @pl.when(pl.program_id(0) == 0)
    def _():
        out_ref[...] = jnp.zeros_like(out_ref)

    lane = jax.lax.broadcasted_iota(jnp.int32, (1, 128), 1)
    out_ref[...] += jnp.where(lane == 0, kl_block, 0.0)


def kernel(student_logits, teacher_logits, labels):
    b, s_len, v = student_logits.shape
    n = b * s_len
    s2 = student_logits.reshape(n, v)
    t2 = teacher_logits.reshape(n, v)
    lab = labels.astype(jnp.int32).reshape(n // _ROWS, 1, _ROWS)

    tk_t, tk_s = _sc_topk(t2, student_logits.reshape(n * v))

    grid = (n // _ROWS,)
    ce_out = pl.pallas_call(
        _tc_ce_kernel,
        grid=grid,
        in_specs=[
            pl.BlockSpec((_ROWS, v), lambda i: (i, 0)),
            pl.BlockSpec((1, 1, _ROWS), lambda i: (i, 0, 0)),
        ],
        out_specs=pl.BlockSpec((1, 128), lambda i: (0, 0)),
        out_shape=jax.ShapeDtypeStruct((1, 128), jnp.float32),
    )(s2, lab)

    kl_rows = 256
    kl_out = pl.pallas_call(
        _tc_kl_kernel,
        grid=(n // kl_rows,),
        in_specs=[
            pl.BlockSpec((kl_rows, _PK), lambda i: (i, 0)),
            pl.BlockSpec((kl_rows, _PK), lambda i: (i, 0)),
        ],
        out_specs=pl.BlockSpec((1, 128), lambda i: (0, 0)),
        out_shape=jax.ShapeDtypeStruct((1, 128), jnp.float32),
    )(tk_t, tk_s)

    n_f = jnp.float32(n)
    ce = ce_out[0, 0] / n_f
    tcs = kl_out[0, 0] / n_f * (_TEMP * _TEMP)
    total = ce + _LAMBDA * tcs
    zero = jnp.zeros((), jnp.float32)
    return (total, ce, tcs, zero)


# speedup vs baseline: 1.3492x; 1.3492x over previous
import functools

import jax
import jax.numpy as jnp
from jax import lax
from jax.experimental import pallas as pl
from jax.experimental.pallas import tpu as pltpu
from jax.experimental.pallas import tpu_sc as plsc

_LAMBDA = 10.0
_TEMP = 5.0
_K = 100
_PK = 128
_ROWS = 16
_NEG = -1.0e30
_CAND_CAP = 32768


def _sortable_u32(x):
    b = plsc.bitcast(x, jnp.uint32)
    sign = jnp.uint32(0x80000000)
    return jnp.where(b >= sign, ~b, b | sign)


def _popcnt(m):
    return plsc.all_reduce_population_count(m)[0]


def _sc_topk(teacher2d, student_flat):
    n, v = teacher2d.shape
    nw = 32
    rows_per_w = n // nw
    nvreg = v // 16

    mesh = plsc.VectorSubcoreMesh(
        core_axis_name="c", subcore_axis_name="s", num_cores=2,
        num_subcores=16)

    @functools.partial(
        pl.kernel,
        out_type=[
            jax.ShapeDtypeStruct((n, _PK), jnp.float32),
            jax.ShapeDtypeStruct((n, _PK), jnp.float32),
        ],
        mesh=mesh,
        compiler_params=pltpu.CompilerParams(needs_layout_passes=False),
        scratch_types=[
            pltpu.VMEM((v,), jnp.float32),
            pltpu.VMEM((_CAND_CAP,), jnp.float32),
            pltpu.VMEM((_CAND_CAP,), jnp.int32),
            pltpu.VMEM((_PK,), jnp.float32),
            pltpu.VMEM((_PK,), jnp.int32),
            pltpu.VMEM((_PK,), jnp.float32),
            pltpu.SemaphoreType.DMA,
        ],
    )
    def topk_kernel(t_hbm, s_hbm, out_t_hbm, out_s_hbm,
                    trow, cand_v, cand_i, outt, outi, outs, sem):
        wid = lax.axis_index("s") * 2 + lax.axis_index("c")
        lane = lax.broadcasted_iota(jnp.int32, (16,), 0)

        def scan_row(probe):
            def body(i, off):
                base = i * 64
                xs = [trow[pl.ds(base + k * 16, 16)] for k in range(4)]
                ms = [x >= probe for x in xs]
                anym = jnp.logical_or(jnp.logical_or(ms[0], ms[1]),
                                      jnp.logical_or(ms[2], ms[3]))

                def slow(off_in):
                    o = off_in
                    for k in range(4):
                        plsc.store_compressed(
                            cand_v.at[pl.ds(o, 16)], xs[k], mask=ms[k])
                        plsc.store_compressed(
                            cand_i.at[pl.ds(o, 16)], lane + (base + k * 16),
                            mask=ms[k])
                        o = o + _popcnt(ms[k])
                    return o

                return lax.cond(_popcnt(anym) > 0, slow, lambda o: o, off)
            return lax.fori_loop(0, nvreg // 4, body, jnp.int32(0))

        def cnt_ge(mid, nv, c):
            def body(j, acc):
                x = cand_v[pl.ds(j * 16, 16)]
                valid = (lane + j * 16) < c
                u = jnp.where(valid, _sortable_u32(x), jnp.uint32(0))
                return acc + jnp.where(u >= mid, 1, 0)
            accv = lax.fori_loop(0, nv, body, jnp.zeros((16,), jnp.int32))
            return plsc.cumsum(accv)[15]

        def do_row(j, _):
            row = wid * rows_per_w + j
            pltpu.sync_copy(t_hbm.at[row], trow)

            c0 = scan_row(jnp.float32(2.45))

            def cond(st):
                return st[2] < _K

            def lower(st):
                probe, step, _ = st
                probe2 = probe - step
                return probe2, step * 4.0, scan_row(probe2)

            probe_f, _, c = lax.while_loop(
                cond, lower, (jnp.float32(2.45), jnp.float32(1.5), c0))

            nv = (c + 15) // 16

            def vmax_body(j, acc):
                x = cand_v[pl.ds(j * 16, 16)]
                valid = (lane + j * 16) < c
                return jnp.maximum(acc, jnp.where(valid, x, -3.4e38))

            mx_vec = lax.fori_loop(
                0, nv, vmax_body, jnp.full((16,), -3.4e38, jnp.float32))
            hi_init = _sortable_u32(plsc.cummax(mx_vec))[15] + jnp.uint32(1)
            lo_init = _sortable_u32(jnp.full((16,), probe_f, jnp.float32))[0]

            def bis_cond(st):
                lo, hi = st
                return hi - lo > jnp.uint32(1)

            def bis(st):
                lo, hi = st
                mid = lo + lax.shift_right_logical(hi - lo, jnp.uint32(1))
                ge = cnt_ge(mid, nv, c) >= _K
                return (jnp.where(ge, mid, lo), jnp.where(ge, hi, mid))

            tau, _ = lax.while_loop(bis_cond, bis, (lo_init, hi_init))
            n_gt = cnt_ge(tau + jnp.uint32(1), nv, c)
            r = _K - n_gt

            def emit(jj, st):
                ooffv, eqrunv = st
                x = cand_v[pl.ds(jj * 16, 16)]
                gi = cand_i[pl.ds(jj * 16, 16)]
                valid = (lane + jj * 16) < c
                u = jnp.where(valid, _sortable_u32(x), jnp.uint32(0))
                m_gt = u > tau
                m_eq = u == tau
                pe = plsc.cumsum(jnp.where(m_eq, 1, 0))
                sel = jnp.logical_or(
                    m_gt, jnp.logical_and(m_eq, pe + eqrunv <= r))
                pfs = plsc.cumsum(jnp.where(sel, 1, 0))
                idx = ooffv + pfs - 1
                plsc.store_scatter(outt, [idx], x, mask=sel)
                plsc.store_scatter(outi, [idx], gi + row * v, mask=sel)
                return (ooffv + plsc.all_reduce_population_count(sel),
                        eqrunv + plsc.all_reduce_population_count(m_eq))

            lax.fori_loop(0, nv, emit, (jnp.zeros((16,), jnp.int32),
                                        jnp.zeros((16,), jnp.int32)))

            p6 = lane + 96
            outt[pl.ds(96, 16)] = jnp.where(
                p6 < _K, outt[pl.ds(96, 16)], jnp.float32(_NEG))
            outi[pl.ds(96, 16)] = jnp.where(
                p6 < _K, outi[pl.ds(96, 16)], row * v)
            outt[pl.ds(112, 16)] = jnp.full((16,), _NEG, jnp.float32)
            outi[pl.ds(112, 16)] = jnp.full((16,), row * v, jnp.int32)

            pltpu.async_copy(s_hbm.at[outi], outs, sem).wait()
            outs[pl.ds(96, 16)] = jnp.where(
                p6 < _K, outs[pl.ds(96, 16)], jnp.float32(_NEG))
            outs[pl.ds(112, 16)] = jnp.full((16,), _NEG, jnp.float32)

            pltpu.sync_copy(outt, out_t_hbm.at[row])
            pltpu.sync_copy(outs, out_s_hbm.at[row])
            return 0

        lax.fori_loop(0, rows_per_w, do_row, 0)

    return topk_kernel(teacher2d, student_flat)


def _tc_ce_kernel(s_ref, lab_ref, out_ref):
    r, v = s_ref.shape
    s = s_ref[...]
    lab = lab_ref[0, 0, :]

    ms = jnp.max(s, axis=1, keepdims=True)
    sum_es = jnp.sum(jnp.exp(s - ms), axis=1, keepdims=True)
    lse = jnp.log(sum_es) + ms
    col = jax.lax.broadcasted_iota(jnp.int32, (r, v), 1)
    lab_logit = jnp.sum(
        jnp.where(col == lab[:, None], s, 0.0), axis=1, keepdims=True)
    nll_block = jnp.sum(lse - lab_logit)

    @pl.when(pl.program_id(0) == 0)
    def _():
        out_ref[...] = jnp.zeros_like(out_ref)

    lane = jax.lax.broadcasted_iota(jnp.int32, (1, 128), 1)
    out_ref[...] += jnp.where(lane == 0, nll_block, 0.0)


def _tc_kl_kernel(tkt_ref, tks_ref, out_ref):
    inv_t = 1.0 / _TEMP
    tk_t = tkt_ref[...]
    tk_s = tks_ref[...]
    m_t = jnp.max(tk_t, axis=1, keepdims=True)
    e = jnp.exp((tk_t - m_t) * inv_t)
    a = jnp.sum(e, axis=1, keepdims=True)
    m_s = jnp.max(tk_s, axis=1, keepdims=True)
    a_s = jnp.sum(jnp.exp((tk_s - m_s) * inv_t), axis=1, keepdims=True)
    u_sum = jnp.sum(e * (tk_t - tk_s), axis=1, keepdims=True) * inv_t
    kl_row = u_sum / a + (m_s - m_t) * inv_t + jnp.log(a_s) - jnp.log(a)
    kl_block = jnp.sum(kl_row)

    @pl.when(pl.program_id(0) == 0)
    def _():
        out_ref[...] = jnp.zeros_like(out_ref)

    lane = jax.lax.broadcasted_iota(jnp.int32, (1, 128), 1)
    out_ref[...] += jnp.where(lane == 0, kl_block, 0.0)


def kernel(student_logits, teacher_logits, labels):
    b, s_len, v = student_logits.shape
    n = b * s_len
    s2 = student_logits.reshape(n, v)
    t2 = teacher_logits.reshape(n, v)
    lab = labels.astype(jnp.int32).reshape(n // _ROWS, 1, _ROWS)

    tk_t, tk_s = _sc_topk(t2, student_logits.reshape(n * v))

    grid = (n // _ROWS,)
    ce_out = pl.pallas_call(
        _tc_ce_kernel,
        grid=grid,
        in_specs=[
            pl.BlockSpec((_ROWS, v), lambda i: (i, 0)),
            pl.BlockSpec((1, 1, _ROWS), lambda i: (i, 0, 0)),
        ],
        out_specs=pl.BlockSpec((1, 128), lambda i: (0, 0)),
        out_shape=jax.ShapeDtypeStruct((1, 128), jnp.float32),
    )(s2, lab)

    kl_rows = 256
    kl_out = pl.pallas_call(
        _tc_kl_kernel,
        grid=(n // kl_rows,),
        in_specs=[
            pl.BlockSpec((kl_rows, _PK), lambda i: (i, 0)),
            pl.BlockSpec((kl_rows, _PK), lambda i: (i, 0)),
        ],
        out_specs=pl.BlockSpec((1, 128), lambda i: (0, 0)),
        out_shape=jax.ShapeDtypeStruct((1, 128), jnp.float32),
    )(tk_t, tk_s)

    n_f = jnp.float32(n)
    ce = ce_out[0, 0] / n_f
    tcs = kl_out[0, 0] / n_f * (_TEMP * _TEMP)
    total = ce + _LAMBDA * tcs
    zero = jnp.zeros((), jnp.float32)
    return (total, ce, tcs, zero)
